# baseline (speedup 1.0000x reference)
import functools
import math

import jax
import jax.numpy as jnp
import numpy as np
from jax.experimental import pallas as pl
from jax.experimental.pallas import tpu as pltpu

T = 2048
HID = 4096
H = 32
D_NOPE = 128
D_ROPE = 64
D_V = 128
Q_LORA = 1536
KV_LORA = 512
THETA = 10000.0

_BF = jnp.bfloat16
_F32 = jnp.float32

M1 = 256
M2 = 256
MQ = 1024
HPB = 2


def _vmem(limit_mb):
    return pltpu.CompilerParams(vmem_limit_bytes=limit_mb * 1024 * 1024)


def _dot(a, b, dims):
    return jax.lax.dot_general(a, b, (dims, ((), ())),
                               preferred_element_type=_F32)


def _pair_swap(x_bf):
    a = jax.lax.broadcasted_iota(jnp.int32, (D_ROPE, D_ROPE), 0)
    b = jax.lax.broadcasted_iota(jnp.int32, (D_ROPE, D_ROPE), 1)
    perm = ((a ^ 1) == b).astype(_BF)
    return _dot(x_bf, perm, ((1,), (0,)))


def _rope(pe32, cos2, sin2):
    swp = _pair_swap(pe32.astype(_BF))
    return pe32 * cos2 + swp * sin2


def _stage1_kernel(x_ref, wqa_ref, wkva_ref, qnw_ref, kvnw_ref, cos_ref,
                   sin_ref, qlat_ref, kvl_ref, kpe_ref):
    x = x_ref[...].astype(_BF)
    xa = _dot(x, wqa_ref[...].astype(_BF), ((1,), (0,)))
    var = jnp.mean(xa * xa, axis=-1, keepdims=True)
    scl = (D_NOPE + D_ROPE) ** -0.5
    qlat_ref[...] = (xa * (jax.lax.rsqrt(var + 1e-6) * scl)
                     * qnw_ref[...]).astype(_BF)
    kv = _dot(x, wkva_ref[...].astype(_BF), ((1,), (0,)))
    kvc = kv[:, :KV_LORA]
    var2 = jnp.mean(kvc * kvc, axis=-1, keepdims=True)
    kvl_ref[...] = (kvc * jax.lax.rsqrt(var2 + 1e-6) * kvnw_ref[...]).astype(_BF)
    pe = kv[:, KV_LORA:]
    kpe_ref[...] = _rope(pe, cos_ref[...], sin_ref[...]).astype(_BF)


def _matmul_kernel(a_ref, w_ref, o_ref):
    w = w_ref[...]
    if w.dtype != _BF:
        w = w.astype(_BF)
    o_ref[...] = _dot(a_ref[...], w, ((1,), (0,))).astype(o_ref.dtype)


def _oproj_kernel(a_ref, w_ref, out_ref):
    k = pl.program_id(0)
    a = a_ref[...]
    NC = 1024
    for n in range(HID // NC):
        w = w_ref[:, n * NC:(n + 1) * NC].astype(_BF)
        part = _dot(a, w, ((1,), (0,)))

        @pl.when(k == 0)
        def _init():
            out_ref[:, n * NC:(n + 1) * NC] = part

        @pl.when(k != 0)
        def _accum():
            out_ref[:, n * NC:(n + 1) * NC] += part


def _attn_kernel(q_ref, kvn_ref, kpe_ref, cos_ref, sin_ref, o_ref,
                 acc_ref, l_ref):
    j = pl.program_id(1)
    qi = (j + 1) // 2
    kj = j // 2
    cos = cos_ref[...]
    sin = sin_ref[...]
    kpe = kpe_ref[...]
    row = jax.lax.broadcasted_iota(jnp.int32, (MQ, MQ), 0)
    col = jax.lax.broadcasted_iota(jnp.int32, (MQ, MQ), 1)
    allow = (row >= col) | (kj != qi)
    for a in range(HPB):
        qa = q_ref[:, a * (D_NOPE + D_ROPE):(a + 1) * (D_NOPE + D_ROPE)]
        pe = qa[:, D_NOPE:].astype(_F32)
        r = _rope(pe, cos, sin).astype(_BF)
        qh = jnp.concatenate([qa[:, :D_NOPE], r], axis=-1)
        kv = kvn_ref[:, a * (D_NOPE + D_V):(a + 1) * (D_NOPE + D_V)]
        kh = jnp.concatenate([kv[:, :D_NOPE], kpe], axis=-1)
        s = _dot(qh, kh, ((1,), (1,)))
        p = jnp.exp(jnp.where(allow, s, -1e30))
        l = jnp.sum(p, axis=-1, keepdims=True)
        pv = _dot(p.astype(_BF), kv[:, D_NOPE:], ((1,), (0,)))

        @pl.when(kj == 0)
        def _init():
            acc_ref[:, a * D_V:(a + 1) * D_V] = pv
            l_ref[:, a:a + 1] = l

        @pl.when(kj != 0)
        def _accum():
            acc_ref[:, a * D_V:(a + 1) * D_V] += pv
            l_ref[:, a:a + 1] += l

    @pl.when(kj == qi)
    def _final():
        for a in range(HPB):
            o_ref[:, a * D_V:(a + 1) * D_V] = (
                acc_ref[:, a * D_V:(a + 1) * D_V] / l_ref[:, a:a + 1]
            ).astype(_BF)


def kernel(x, positions, wq_a, q_norm_w, wq_b, wkv_a, kv_norm_w, wkv_b, wo):
    pos_f = positions.astype(_F32)
    inv_freq = 1.0 / (THETA ** (jnp.arange(0, D_ROPE, 2, dtype=_F32) / D_ROPE))
    ang = pos_f[:, None] * inv_freq[None, :]
    cos = jnp.cos(ang)
    sin = jnp.sin(ang)
    cos2 = jnp.repeat(cos, 2, axis=1)
    sin2 = jnp.stack([-sin, sin], axis=-1).reshape(T, D_ROPE)

    qnw2 = q_norm_w.reshape(1, Q_LORA)
    kvnw2 = kv_norm_w.reshape(1, KV_LORA)

    qlat, kvl, kpe = pl.pallas_call(
        _stage1_kernel,
        grid=(T // M1,),
        in_specs=[
            pl.BlockSpec((M1, HID), lambda i: (i, 0)),
            pl.BlockSpec((HID, Q_LORA), lambda i: (0, 0)),
            pl.BlockSpec((HID, KV_LORA + D_ROPE), lambda i: (0, 0)),
            pl.BlockSpec((1, Q_LORA), lambda i: (0, 0)),
            pl.BlockSpec((1, KV_LORA), lambda i: (0, 0)),
            pl.BlockSpec((M1, D_ROPE), lambda i: (i, 0)),
            pl.BlockSpec((M1, D_ROPE), lambda i: (i, 0)),
        ],
        out_specs=[
            pl.BlockSpec((M1, Q_LORA), lambda i: (i, 0)),
            pl.BlockSpec((M1, KV_LORA), lambda i: (i, 0)),
            pl.BlockSpec((M1, D_ROPE), lambda i: (i, 0)),
        ],
        out_shape=[
            jax.ShapeDtypeStruct((T, Q_LORA), _BF),
            jax.ShapeDtypeStruct((T, KV_LORA), _BF),
            jax.ShapeDtypeStruct((T, D_ROPE), _BF),
        ],
        compiler_params=_vmem(60),
    )(x, wq_a, wkv_a, qnw2, kvnw2, cos2, sin2)

    q = pl.pallas_call(
        _matmul_kernel,
        grid=(T // M2,),
        in_specs=[
            pl.BlockSpec((M2, Q_LORA), lambda i: (i, 0)),
            pl.BlockSpec((Q_LORA, H * (D_NOPE + D_ROPE)), lambda i: (0, 0)),
        ],
        out_specs=pl.BlockSpec((M2, H * (D_NOPE + D_ROPE)), lambda i: (i, 0)),
        out_shape=jax.ShapeDtypeStruct((T, H * (D_NOPE + D_ROPE)), _BF),
        compiler_params=_vmem(60),
    )(qlat, wq_b)

    kvn = pl.pallas_call(
        _matmul_kernel,
        grid=(T // M2,),
        in_specs=[
            pl.BlockSpec((M2, KV_LORA), lambda i: (i, 0)),
            pl.BlockSpec((KV_LORA, H * (D_NOPE + D_V)), lambda i: (0, 0)),
        ],
        out_specs=pl.BlockSpec((M2, H * (D_NOPE + D_V)), lambda i: (i, 0)),
        out_shape=jax.ShapeDtypeStruct((T, H * (D_NOPE + D_V)), _BF),
        compiler_params=_vmem(56),
    )(kvl, wkv_b)

    o = pl.pallas_call(
        _attn_kernel,
        grid=(H // HPB, 3),
        in_specs=[
            pl.BlockSpec((MQ, HPB * (D_NOPE + D_ROPE)),
                         lambda h, j: ((j + 1) // 2, h)),
            pl.BlockSpec((MQ, HPB * (D_NOPE + D_V)), lambda h, j: (j // 2, h)),
            pl.BlockSpec((MQ, D_ROPE), lambda h, j: (j // 2, 0)),
            pl.BlockSpec((MQ, D_ROPE), lambda h, j: ((j + 1) // 2, 0)),
            pl.BlockSpec((MQ, D_ROPE), lambda h, j: ((j + 1) // 2, 0)),
        ],
        out_specs=pl.BlockSpec((MQ, HPB * D_V), lambda h, j: ((j + 1) // 2, h)),
        out_shape=jax.ShapeDtypeStruct((T, H * D_V), _BF),
        scratch_shapes=[
            pltpu.VMEM((MQ, HPB * D_V), _F32),
            pltpu.VMEM((MQ, HPB), _F32),
        ],
        compiler_params=_vmem(56),
    )(q, kvn, kpe, cos2, sin2)

    KO = 256
    out = pl.pallas_call(
        _oproj_kernel,
        grid=(H * D_V // KO,),
        in_specs=[
            pl.BlockSpec((T, KO), lambda k: (0, k)),
            pl.BlockSpec((KO, HID), lambda k: (k, 0)),
        ],
        out_specs=pl.BlockSpec((T, HID), lambda k: (0, 0)),
        out_shape=jax.ShapeDtypeStruct((T, HID), _F32),
        compiler_params=_vmem(62),
    )(o, wo)

    return out


# speedup vs baseline: 1.0686x; 1.0686x over previous
import functools
import math

import jax
import jax.numpy as jnp
import numpy as np
from jax.experimental import pallas as pl
from jax.experimental.pallas import tpu as pltpu

T = 2048
HID = 4096
H = 32
D_NOPE = 128
D_ROPE = 64
D_V = 128
Q_LORA = 1536
KV_LORA = 512
THETA = 10000.0

_BF = jnp.bfloat16
_F32 = jnp.float32

M1 = 256
M2 = 256
MQ = 1024
HPB = 2


def _vmem(limit_mb):
    return pltpu.CompilerParams(vmem_limit_bytes=limit_mb * 1024 * 1024)


def _dot(a, b, dims):
    return jax.lax.dot_general(a, b, (dims, ((), ())),
                               preferred_element_type=_F32)


def _pair_swap(x_bf):
    a = jax.lax.broadcasted_iota(jnp.int32, (D_ROPE, D_ROPE), 0)
    b = jax.lax.broadcasted_iota(jnp.int32, (D_ROPE, D_ROPE), 1)
    perm = ((a ^ 1) == b).astype(_BF)
    return _dot(x_bf, perm, ((1,), (0,)))


def _rope(pe32, cos2, sin2):
    swp = _pair_swap(pe32.astype(_BF))
    return pe32 * cos2 + swp * sin2


def _stage1_kernel(x_ref, wqa_ref, wkva_ref, qnw_ref, kvnw_ref, cos_ref,
                   sin_ref, qlat_ref, kvl_ref, kpe_ref):
    x = x_ref[...].astype(_BF)
    xa = _dot(x, wqa_ref[...].astype(_BF), ((1,), (0,)))
    var = jnp.mean(xa * xa, axis=-1, keepdims=True)
    scl = (D_NOPE + D_ROPE) ** -0.5
    qlat_ref[...] = (xa * (jax.lax.rsqrt(var + 1e-6) * scl)
                     * qnw_ref[...]).astype(_BF)
    kv = _dot(x, wkva_ref[...].astype(_BF), ((1,), (0,)))
    kvc = kv[:, :KV_LORA]
    var2 = jnp.mean(kvc * kvc, axis=-1, keepdims=True)
    kvl_ref[...] = (kvc * jax.lax.rsqrt(var2 + 1e-6) * kvnw_ref[...]).astype(_BF)
    pe = kv[:, KV_LORA:]
    kpe_ref[...] = _rope(pe, cos_ref[...], sin_ref[...]).astype(_BF)


def _matmul_kernel(a_ref, w_ref, o_ref):
    w = w_ref[...]
    if w.dtype != _BF:
        w = w.astype(_BF)
    o_ref[...] = _dot(a_ref[...], w, ((1,), (0,))).astype(o_ref.dtype)


def _oproj_kernel(a_ref, w_ref, out_ref):
    out_ref[...] = _dot(a_ref[...], w_ref[...], ((1,), (0,)))


def _attn_kernel(q_ref, kvn_ref, kpe_ref, cos_ref, sin_ref, o_ref,
                 acc_ref, l_ref):
    j = pl.program_id(1)
    qi = (j + 1) // 2
    kj = j // 2
    cos = cos_ref[...]
    sin = sin_ref[...]
    kpe = kpe_ref[...]
    row = jax.lax.broadcasted_iota(jnp.int32, (MQ, MQ), 0)
    col = jax.lax.broadcasted_iota(jnp.int32, (MQ, MQ), 1)
    allow = (row >= col) | (kj != qi)
    for a in range(HPB):
        qa = q_ref[:, a * (D_NOPE + D_ROPE):(a + 1) * (D_NOPE + D_ROPE)]
        pe = qa[:, D_NOPE:].astype(_F32)
        r = _rope(pe, cos, sin).astype(_BF)
        qh = jnp.concatenate([qa[:, :D_NOPE], r], axis=-1)
        kv = kvn_ref[:, a * (D_NOPE + D_V):(a + 1) * (D_NOPE + D_V)]
        kh = jnp.concatenate([kv[:, :D_NOPE], kpe], axis=-1)
        s = _dot(qh, kh, ((1,), (1,)))
        p = jnp.exp(jnp.where(allow, s, -1e30))
        l = jnp.sum(p, axis=-1, keepdims=True)
        pv = _dot(p.astype(_BF), kv[:, D_NOPE:], ((1,), (0,)))

        @pl.when(kj == 0)
        def _init():
            acc_ref[:, a * D_V:(a + 1) * D_V] = pv
            l_ref[:, a:a + 1] = l

        @pl.when(kj != 0)
        def _accum():
            acc_ref[:, a * D_V:(a + 1) * D_V] += pv
            l_ref[:, a:a + 1] += l

    @pl.when(kj == qi)
    def _final():
        for a in range(HPB):
            o_ref[:, a * D_V:(a + 1) * D_V] = (
                acc_ref[:, a * D_V:(a + 1) * D_V] / l_ref[:, a:a + 1]
            ).astype(_BF)


def kernel(x, positions, wq_a, q_norm_w, wq_b, wkv_a, kv_norm_w, wkv_b, wo):
    pos_f = positions.astype(_F32)
    inv_freq = 1.0 / (THETA ** (jnp.arange(0, D_ROPE, 2, dtype=_F32) / D_ROPE))
    ang = pos_f[:, None] * inv_freq[None, :]
    cos = jnp.cos(ang)
    sin = jnp.sin(ang)
    cos2 = jnp.repeat(cos, 2, axis=1)
    sin2 = jnp.stack([-sin, sin], axis=-1).reshape(T, D_ROPE)

    qnw2 = q_norm_w.reshape(1, Q_LORA)
    kvnw2 = kv_norm_w.reshape(1, KV_LORA)

    qlat, kvl, kpe = pl.pallas_call(
        _stage1_kernel,
        grid=(T // M1,),
        in_specs=[
            pl.BlockSpec((M1, HID), lambda i: (i, 0)),
            pl.BlockSpec((HID, Q_LORA), lambda i: (0, 0)),
            pl.BlockSpec((HID, KV_LORA + D_ROPE), lambda i: (0, 0)),
            pl.BlockSpec((1, Q_LORA), lambda i: (0, 0)),
            pl.BlockSpec((1, KV_LORA), lambda i: (0, 0)),
            pl.BlockSpec((M1, D_ROPE), lambda i: (i, 0)),
            pl.BlockSpec((M1, D_ROPE), lambda i: (i, 0)),
        ],
        out_specs=[
            pl.BlockSpec((M1, Q_LORA), lambda i: (i, 0)),
            pl.BlockSpec((M1, KV_LORA), lambda i: (i, 0)),
            pl.BlockSpec((M1, D_ROPE), lambda i: (i, 0)),
        ],
        out_shape=[
            jax.ShapeDtypeStruct((T, Q_LORA), _BF),
            jax.ShapeDtypeStruct((T, KV_LORA), _BF),
            jax.ShapeDtypeStruct((T, D_ROPE), _BF),
        ],
        compiler_params=_vmem(60),
    )(x, wq_a, wkv_a, qnw2, kvnw2, cos2, sin2)

    q = pl.pallas_call(
        _matmul_kernel,
        grid=(T // M2,),
        in_specs=[
            pl.BlockSpec((M2, Q_LORA), lambda i: (i, 0)),
            pl.BlockSpec((Q_LORA, H * (D_NOPE + D_ROPE)), lambda i: (0, 0)),
        ],
        out_specs=pl.BlockSpec((M2, H * (D_NOPE + D_ROPE)), lambda i: (i, 0)),
        out_shape=jax.ShapeDtypeStruct((T, H * (D_NOPE + D_ROPE)), _BF),
        compiler_params=_vmem(60),
    )(qlat, wq_b)

    kvn = pl.pallas_call(
        _matmul_kernel,
        grid=(T // M2,),
        in_specs=[
            pl.BlockSpec((M2, KV_LORA), lambda i: (i, 0)),
            pl.BlockSpec((KV_LORA, H * (D_NOPE + D_V)), lambda i: (0, 0)),
        ],
        out_specs=pl.BlockSpec((M2, H * (D_NOPE + D_V)), lambda i: (i, 0)),
        out_shape=jax.ShapeDtypeStruct((T, H * (D_NOPE + D_V)), _BF),
        compiler_params=_vmem(56),
    )(kvl, wkv_b)

    o = pl.pallas_call(
        _attn_kernel,
        grid=(H // HPB, 3),
        in_specs=[
            pl.BlockSpec((MQ, HPB * (D_NOPE + D_ROPE)),
                         lambda h, j: ((j + 1) // 2, h)),
            pl.BlockSpec((MQ, HPB * (D_NOPE + D_V)), lambda h, j: (j // 2, h)),
            pl.BlockSpec((MQ, D_ROPE), lambda h, j: (j // 2, 0)),
            pl.BlockSpec((MQ, D_ROPE), lambda h, j: ((j + 1) // 2, 0)),
            pl.BlockSpec((MQ, D_ROPE), lambda h, j: ((j + 1) // 2, 0)),
        ],
        out_specs=pl.BlockSpec((MQ, HPB * D_V), lambda h, j: ((j + 1) // 2, h)),
        out_shape=jax.ShapeDtypeStruct((T, H * D_V), _BF),
        scratch_shapes=[
            pltpu.VMEM((MQ, HPB * D_V), _F32),
            pltpu.VMEM((MQ, HPB), _F32),
        ],
        compiler_params=_vmem(56),
    )(q, kvn, kpe, cos2, sin2)

    M5 = 512
    out = pl.pallas_call(
        _oproj_kernel,
        grid=(T // M5,),
        in_specs=[
            pl.BlockSpec((M5, H * D_V), lambda i: (i, 0)),
            pl.BlockSpec((H * D_V, HID), lambda i: (0, 0)),
        ],
        out_specs=pl.BlockSpec((M5, HID), lambda i: (i, 0)),
        out_shape=jax.ShapeDtypeStruct((T, HID), _F32),
        compiler_params=_vmem(60),
    )(o, wo.astype(_BF))

    return out
